# baseline (speedup 1.0000x reference)
import jax
import jax.numpy as jnp
from jax import lax
from jax.experimental import pallas as pl
from jax.experimental.pallas import tpu as pltpu
from jax.experimental.pallas import tpu_sc as plsc

N = 10000
E = 320000
F_IN = 128
HID = 256
C = 3
G = 64

NC = 2
NS = 16

NPAD = 10240
ROWS_PER_TILE = NPAD // NS
CH = 256
IB = 8
NCH = 80
NGRP = NCH // IB
EP = NS * NCH * CH
ED = EP // (NC * NS)
RCH = 128
HALF = HID // 2

TR = 2048
NG = NPAD // TR


def _mesh():
    return plsc.VectorSubcoreMesh(
        core_axis_name="c", subcore_axis_name="s", num_cores=NC, num_subcores=NS
    )


def _deg_body(dst_hbm, deg_out, dst_v, ones_v, row_v, deg_sh):
    c = lax.axis_index("c")
    s = lax.axis_index("s")
    pltpu.sync_copy(dst_hbm.at[c, s], dst_v)

    ones16 = jnp.ones((16,), jnp.float32)

    def fill(i, _):
        ones_v[pl.ds(i * 16, 16)] = ones16
        return _

    lax.fori_loop(0, ED // 16, fill, 0)

    zero16 = jnp.zeros((16,), jnp.float32)

    def zfill(i, _):
        row_v[pl.ds(i * 16, 16)] = zero16
        return _

    lax.fori_loop(0, ROWS_PER_TILE // 16, zfill, 0)

    row_lo = s * ROWS_PER_TILE
    pltpu.sync_copy(row_v, deg_sh.at[pl.ds(row_lo, ROWS_PER_TILE)])
    plsc.subcore_barrier()
    pltpu.sync_copy(ones_v, deg_sh.at[dst_v], add=True)
    plsc.subcore_barrier()
    pltpu.sync_copy(deg_sh.at[pl.ds(row_lo, ROWS_PER_TILE)], row_v)
    pltpu.sync_copy(row_v, deg_out.at[c, pl.ds(row_lo, ROWS_PER_TILE)])


def _deg_call(dst_d):
    return pl.kernel(
        _deg_body,
        out_type=jax.ShapeDtypeStruct((NC, NPAD), jnp.float32),
        mesh=_mesh(),
        scratch_types=[
            pltpu.VMEM((ED,), jnp.int32),
            pltpu.VMEM((ED,), jnp.float32),
            pltpu.VMEM((ROWS_PER_TILE,), jnp.float32),
            pltpu.VMEM_SHARED((NPAD,), jnp.float32),
        ],
    )(dst_d)


def _agg_body(src_hbm, dst_hbm, u_hbm, acc_hbm, src_v, dst_v, rows_a,
              acc_sh, sem_a):
    c = lax.axis_index("c")
    s = lax.axis_index("s")
    row_lo = s * ROWS_PER_TILE

    def run(uc, oc):
        for k in range(ROWS_PER_TILE // RCH):
            pltpu.sync_copy(uc.at[pl.ds(row_lo + k * RCH, RCH)],
                            rows_a.at[pl.ds(0, RCH)])
            pltpu.sync_copy(rows_a.at[pl.ds(0, RCH)],
                            acc_sh.at[pl.ds(row_lo + k * RCH, RCH)])
        plsc.subcore_barrier()

        def grp(g, _):
            pltpu.sync_copy(src_hbm.at[s, pl.ds(g * IB * CH, IB * CH)], src_v)
            pltpu.sync_copy(dst_hbm.at[s, pl.ds(g * IB * CH, IB * CH)], dst_v)

            def body(j, _2):
                pltpu.async_copy(
                    uc.at[src_v.at[pl.ds(j * CH, CH)]], rows_a, sem_a).wait()
                pltpu.sync_copy(rows_a, acc_sh.at[dst_v.at[pl.ds(j * CH, CH)]],
                                add=True)
                return _2

            return lax.fori_loop(0, IB, body, _)

        lax.fori_loop(0, NGRP, grp, 0)
        plsc.subcore_barrier()
        for k in range(ROWS_PER_TILE // RCH):
            pltpu.sync_copy(acc_sh.at[pl.ds(row_lo + k * RCH, RCH)],
                            rows_a.at[pl.ds(0, RCH)])
            pltpu.sync_copy(rows_a.at[pl.ds(0, RCH)],
                            oc.at[pl.ds(row_lo + k * RCH, RCH)])

    pl.when(c == 0)(lambda: run(u_hbm.at[0], acc_hbm.at[0]))
    pl.when(c == 1)(lambda: run(u_hbm.at[1], acc_hbm.at[1]))


def _agg_call(src_r, dst_r, u):
    return pl.kernel(
        _agg_body,
        out_type=jax.ShapeDtypeStruct((NC, NPAD, HALF), jnp.float32),
        mesh=_mesh(),
        scratch_types=[
            pltpu.VMEM((IB * CH,), jnp.int32),
            pltpu.VMEM((IB * CH,), jnp.int32),
            pltpu.VMEM((CH, HALF), jnp.float32),
            pltpu.VMEM_SHARED((NPAD, HALF), jnp.float32),
            pltpu.SemaphoreType.DMA,
        ],
    )(src_r, dst_r, u)


def _dinv_of(deg_ref):
    dsum = jnp.sum(deg_ref[...], axis=1, keepdims=True) + 1.0
    return lax.rsqrt(dsum)


def _mm1_body(x_ref, w_ref, deg_ref, u_ref):
    dinv = _dinv_of(deg_ref)
    h = jnp.dot(x_ref[...], w_ref[...], preferred_element_type=jnp.float32)
    u = h * dinv
    u_ref[0] = u[:, :HALF]
    u_ref[1] = u[:, HALF:]


def _mm1_call(x_p, W1, deg_t):
    return pl.pallas_call(
        _mm1_body,
        grid=(NG,),
        in_specs=[
            pl.BlockSpec((TR, F_IN), lambda r: (r, 0)),
            pl.BlockSpec((F_IN, HID), lambda r: (0, 0)),
            pl.BlockSpec((TR, NC), lambda r: (r, 0)),
        ],
        out_specs=pl.BlockSpec((NC, TR, HALF), lambda r: (0, r, 0)),
        out_shape=jax.ShapeDtypeStruct((NC, NPAD, HALF), jnp.float32),
    )(x_p, W1, deg_t)


def _mm2_body(acc_ref, w_ref, b_ref, deg_ref, u_ref):
    dinv = _dinv_of(deg_ref)
    hcat = jnp.concatenate([acc_ref[0], acc_ref[1]], axis=1)
    x2 = jnp.maximum(hcat * dinv + b_ref[...], 0.0)
    h2 = jnp.dot(x2, w_ref[...], preferred_element_type=jnp.float32)
    u = h2 * dinv
    u_ref[0] = u[:, :HALF]
    u_ref[1] = u[:, HALF:]


def _mm2_call(acc1, W2, b1_2d, deg_t):
    return pl.pallas_call(
        _mm2_body,
        grid=(NG,),
        in_specs=[
            pl.BlockSpec((NC, TR, HALF), lambda r: (0, r, 0)),
            pl.BlockSpec((HID, HID), lambda r: (0, 0)),
            pl.BlockSpec((1, HID), lambda r: (0, 0)),
            pl.BlockSpec((TR, NC), lambda r: (r, 0)),
        ],
        out_specs=pl.BlockSpec((NC, TR, HALF), lambda r: (0, r, 0)),
        out_shape=jax.ShapeDtypeStruct((NC, NPAD, HALF), jnp.float32),
    )(acc1, W2, b1_2d, deg_t)


def _final_body(acc_ref, deg_ref, b_ref, batch_ref, wc_ref, bc_ref, out_ref,
                pooled_acc, cnt_acc):
    r = pl.program_id(0)

    @pl.when(r == 0)
    def _init():
        pooled_acc[...] = jnp.zeros((G, HID), jnp.float32)
        cnt_acc[...] = jnp.zeros((G, HID), jnp.float32)

    dinv = _dinv_of(deg_ref)
    hcat = jnp.concatenate([acc_ref[0], acc_ref[1]], axis=1)
    h = hcat * dinv
    b = batch_ref[0]
    oh = (lax.broadcasted_iota(jnp.int32, (G, TR), 0) == b).astype(jnp.float32)
    pooled_acc[...] += jnp.dot(oh, h, preferred_element_type=jnp.float32)
    cnt_acc[...] += jnp.broadcast_to(
        jnp.sum(oh, axis=1, keepdims=True), (G, HID)
    )

    @pl.when(r == NG - 1)
    def _fin():
        cnt = cnt_acc[...]
        sums = pooled_acc[...] + cnt * b_ref[...]
        mean = sums / jnp.maximum(cnt, 1.0)
        out_ref[...] = (
            jnp.dot(mean, wc_ref[...], preferred_element_type=jnp.float32)
            + bc_ref[...]
        )


def _final_call(acc2, deg_t, b2_2d, batch_r, Wc_p, bc_p):
    return pl.pallas_call(
        _final_body,
        grid=(NG,),
        in_specs=[
            pl.BlockSpec((NC, TR, HALF), lambda r: (0, r, 0)),
            pl.BlockSpec((TR, NC), lambda r: (r, 0)),
            pl.BlockSpec((1, HID), lambda r: (0, 0)),
            pl.BlockSpec((1, 1, TR), lambda r: (r, 0, 0)),
            pl.BlockSpec((HID, 128), lambda r: (0, 0)),
            pl.BlockSpec((1, 128), lambda r: (0, 0)),
        ],
        out_specs=pl.BlockSpec((G, 128), lambda r: (0, 0)),
        out_shape=jax.ShapeDtypeStruct((G, 128), jnp.float32),
        scratch_shapes=[
            pltpu.VMEM((G, HID), jnp.float32),
            pltpu.VMEM((G, HID), jnp.float32),
        ],
    )(acc2, deg_t, b2_2d, batch_r, Wc_p, bc_p)


def kernel(x, edge_index, batch, W1, b1, W2, b2, Wc, bc):
    x_p = jnp.pad(x, ((0, NPAD - N), (0, 0)))
    src = jnp.pad(edge_index[0], (0, EP - E))
    dst = jnp.pad(edge_index[1], (0, EP - E), constant_values=N)
    src_r = src.reshape(NS, NCH * CH)
    dst_r = dst.reshape(NS, NCH * CH)
    dst_d = dst.reshape(NC, NS, ED)

    deg_p = _deg_call(dst_d)
    deg_t = deg_p.T

    u1 = _mm1_call(x_p, W1, deg_t)
    acc1 = _agg_call(src_r, dst_r, u1)
    u2 = _mm2_call(acc1, W2, jnp.reshape(b1, (1, HID)), deg_t)
    acc2 = _agg_call(src_r, dst_r, u2)

    batch_r = jnp.pad(batch, (0, NPAD - N), constant_values=G).reshape(NG, 1, TR)
    Wc_p = jnp.pad(Wc, ((0, 0), (0, 128 - C)))
    bc_p = jnp.reshape(jnp.pad(bc, (0, 128 - C)), (1, 128))

    out_p = _final_call(acc2, deg_t, jnp.reshape(b2, (1, HID)), batch_r, Wc_p, bc_p)
    return out_p[:, :C]


# speedup vs baseline: 1.0652x; 1.0652x over previous
import jax
import jax.numpy as jnp
from jax import lax
from jax.experimental import pallas as pl
from jax.experimental.pallas import tpu as pltpu
from jax.experimental.pallas import tpu_sc as plsc

N = 10000
E = 320000
F_IN = 128
HID = 256
C = 3
G = 64

NC = 2
NS = 16

NPAD = 10240
ROWS_PER_TILE = NPAD // NS
CH = 64
IB = 16
NCH = 320
NGRP = NCH // IB
EP = NS * NCH * CH
ED = EP // (NC * NS)
RCH = 64
HALF = HID // 2

TR = 2048
NG = NPAD // TR


def _mesh():
    return plsc.VectorSubcoreMesh(
        core_axis_name="c", subcore_axis_name="s", num_cores=NC, num_subcores=NS
    )


def _deg_body(dst_hbm, deg_out, dst_v, ones_v, row_v, deg_sh):
    c = lax.axis_index("c")
    s = lax.axis_index("s")
    pltpu.sync_copy(dst_hbm.at[c, s], dst_v)

    ones16 = jnp.ones((16,), jnp.float32)

    def fill(i, _):
        ones_v[pl.ds(i * 16, 16)] = ones16
        return _

    lax.fori_loop(0, ED // 16, fill, 0)

    zero16 = jnp.zeros((16,), jnp.float32)

    def zfill(i, _):
        row_v[pl.ds(i * 16, 16)] = zero16
        return _

    lax.fori_loop(0, ROWS_PER_TILE // 16, zfill, 0)

    row_lo = s * ROWS_PER_TILE
    pltpu.sync_copy(row_v, deg_sh.at[pl.ds(row_lo, ROWS_PER_TILE)])
    plsc.subcore_barrier()
    pltpu.sync_copy(ones_v, deg_sh.at[dst_v], add=True)
    plsc.subcore_barrier()
    pltpu.sync_copy(deg_sh.at[pl.ds(row_lo, ROWS_PER_TILE)], row_v)
    pltpu.sync_copy(row_v, deg_out.at[c, pl.ds(row_lo, ROWS_PER_TILE)])


def _deg_call(dst_d):
    return pl.kernel(
        _deg_body,
        out_type=jax.ShapeDtypeStruct((NC, NPAD), jnp.float32),
        mesh=_mesh(),
        scratch_types=[
            pltpu.VMEM((ED,), jnp.int32),
            pltpu.VMEM((ED,), jnp.float32),
            pltpu.VMEM((ROWS_PER_TILE,), jnp.float32),
            pltpu.VMEM_SHARED((NPAD,), jnp.float32),
        ],
    )(dst_d)


def _agg_body(src_hbm, dst_hbm, u_hbm, acc_hbm, src_v, dst_v,
              r0, r1, r2, r3, acc_sh,
              g0, g1, g2, g3, s0, s1, s2, s3):
    c = lax.axis_index("c")
    s = lax.axis_index("s")
    row_lo = s * ROWS_PER_TILE
    bufs = (r0, r1, r2, r3)
    gsems = (g0, g1, g2, g3)
    ssems = (s0, s1, s2, s3)

    def run(uc, oc):
        for k in range(ROWS_PER_TILE // RCH):
            pltpu.sync_copy(uc.at[pl.ds(row_lo + k * RCH, RCH)], r0.at[pl.ds(0, RCH)])
            pltpu.sync_copy(r0.at[pl.ds(0, RCH)],
                            acc_sh.at[pl.ds(row_lo + k * RCH, RCH)])
        plsc.subcore_barrier()

        def grp(g, _):
            pltpu.sync_copy(src_hbm.at[s, pl.ds(g * IB * CH, IB * CH)], src_v)
            pltpu.sync_copy(dst_hbm.at[s, pl.ds(g * IB * CH, IB * CH)], dst_v)
            for b in range(4):
                pltpu.async_copy(uc.at[src_v.at[pl.ds(b * CH, CH)]],
                                 bufs[b], gsems[b])

            def quad(q, _2):
                j0 = 4 * q
                for b in range(4):
                    pltpu.make_async_copy(
                        uc.at[src_v.at[pl.ds((j0 + b) * CH, CH)]],
                        bufs[b], gsems[b]).wait()
                    pltpu.async_copy(
                        bufs[b], acc_sh.at[dst_v.at[pl.ds((j0 + b) * CH, CH)]],
                        ssems[b], add=True)
                for b in range(4):
                    pltpu.make_async_copy(
                        bufs[b], acc_sh.at[dst_v.at[pl.ds((j0 + b) * CH, CH)]],
                        ssems[b]).wait()

                    @pl.when(j0 + 4 + b < IB)
                    def _pref():
                        pltpu.async_copy(
                            uc.at[src_v.at[pl.ds((j0 + 4 + b) * CH, CH)]],
                            bufs[b], gsems[b])

                return _2

            return lax.fori_loop(0, IB // 4, quad, _)

        lax.fori_loop(0, NGRP, grp, 0)
        plsc.subcore_barrier()
        for k in range(ROWS_PER_TILE // RCH):
            pltpu.sync_copy(acc_sh.at[pl.ds(row_lo + k * RCH, RCH)],
                            r0.at[pl.ds(0, RCH)])
            pltpu.sync_copy(r0.at[pl.ds(0, RCH)], oc.at[pl.ds(row_lo + k * RCH, RCH)])

    pl.when(c == 0)(lambda: run(u_hbm.at[0], acc_hbm.at[0]))
    pl.when(c == 1)(lambda: run(u_hbm.at[1], acc_hbm.at[1]))


def _agg_call(src_r, dst_r, u):
    return pl.kernel(
        _agg_body,
        out_type=jax.ShapeDtypeStruct((NC, NPAD, HALF), jnp.float32),
        mesh=_mesh(),
        scratch_types=[
            pltpu.VMEM((IB * CH,), jnp.int32),
            pltpu.VMEM((IB * CH,), jnp.int32),
            pltpu.VMEM((CH, HALF), jnp.float32),
            pltpu.VMEM((CH, HALF), jnp.float32),
            pltpu.VMEM((CH, HALF), jnp.float32),
            pltpu.VMEM((CH, HALF), jnp.float32),
            pltpu.VMEM_SHARED((NPAD, HALF), jnp.float32),
            pltpu.SemaphoreType.DMA,
            pltpu.SemaphoreType.DMA,
            pltpu.SemaphoreType.DMA,
            pltpu.SemaphoreType.DMA,
            pltpu.SemaphoreType.DMA,
            pltpu.SemaphoreType.DMA,
            pltpu.SemaphoreType.DMA,
            pltpu.SemaphoreType.DMA,
        ],
    )(src_r, dst_r, u)


def _dinv_of(deg_ref):
    dsum = jnp.sum(deg_ref[...], axis=1, keepdims=True) + 1.0
    return lax.rsqrt(dsum)


def _mm1_body(x_ref, w_ref, deg_ref, u_ref):
    dinv = _dinv_of(deg_ref)
    h = jnp.dot(x_ref[...], w_ref[...], preferred_element_type=jnp.float32)
    u = h * dinv
    u_ref[0] = u[:, :HALF]
    u_ref[1] = u[:, HALF:]


def _mm1_call(x_p, W1, deg_t):
    return pl.pallas_call(
        _mm1_body,
        grid=(NG,),
        in_specs=[
            pl.BlockSpec((TR, F_IN), lambda r: (r, 0)),
            pl.BlockSpec((F_IN, HID), lambda r: (0, 0)),
            pl.BlockSpec((TR, NC), lambda r: (r, 0)),
        ],
        out_specs=pl.BlockSpec((NC, TR, HALF), lambda r: (0, r, 0)),
        out_shape=jax.ShapeDtypeStruct((NC, NPAD, HALF), jnp.float32),
    )(x_p, W1, deg_t)


def _mm2_body(acc_ref, w_ref, b_ref, deg_ref, u_ref):
    dinv = _dinv_of(deg_ref)
    hcat = jnp.concatenate([acc_ref[0], acc_ref[1]], axis=1)
    x2 = jnp.maximum(hcat * dinv + b_ref[...], 0.0)
    h2 = jnp.dot(x2, w_ref[...], preferred_element_type=jnp.float32)
    u = h2 * dinv
    u_ref[0] = u[:, :HALF]
    u_ref[1] = u[:, HALF:]


def _mm2_call(acc1, W2, b1_2d, deg_t):
    return pl.pallas_call(
        _mm2_body,
        grid=(NG,),
        in_specs=[
            pl.BlockSpec((NC, TR, HALF), lambda r: (0, r, 0)),
            pl.BlockSpec((HID, HID), lambda r: (0, 0)),
            pl.BlockSpec((1, HID), lambda r: (0, 0)),
            pl.BlockSpec((TR, NC), lambda r: (r, 0)),
        ],
        out_specs=pl.BlockSpec((NC, TR, HALF), lambda r: (0, r, 0)),
        out_shape=jax.ShapeDtypeStruct((NC, NPAD, HALF), jnp.float32),
    )(acc1, W2, b1_2d, deg_t)


def _final_body(acc_ref, deg_ref, b_ref, batch_ref, wc_ref, bc_ref, out_ref,
                pooled_acc, cnt_acc):
    r = pl.program_id(0)

    @pl.when(r == 0)
    def _init():
        pooled_acc[...] = jnp.zeros((G, HID), jnp.float32)
        cnt_acc[...] = jnp.zeros((G, HID), jnp.float32)

    dinv = _dinv_of(deg_ref)
    hcat = jnp.concatenate([acc_ref[0], acc_ref[1]], axis=1)
    h = hcat * dinv
    b = batch_ref[0]
    oh = (lax.broadcasted_iota(jnp.int32, (G, TR), 0) == b).astype(jnp.float32)
    pooled_acc[...] += jnp.dot(oh, h, preferred_element_type=jnp.float32)
    cnt_acc[...] += jnp.broadcast_to(
        jnp.sum(oh, axis=1, keepdims=True), (G, HID)
    )

    @pl.when(r == NG - 1)
    def _fin():
        cnt = cnt_acc[...]
        sums = pooled_acc[...] + cnt * b_ref[...]
        mean = sums / jnp.maximum(cnt, 1.0)
        out_ref[...] = (
            jnp.dot(mean, wc_ref[...], preferred_element_type=jnp.float32)
            + bc_ref[...]
        )


def _final_call(acc2, deg_t, b2_2d, batch_r, Wc_p, bc_p):
    return pl.pallas_call(
        _final_body,
        grid=(NG,),
        in_specs=[
            pl.BlockSpec((NC, TR, HALF), lambda r: (0, r, 0)),
            pl.BlockSpec((TR, NC), lambda r: (r, 0)),
            pl.BlockSpec((1, HID), lambda r: (0, 0)),
            pl.BlockSpec((1, 1, TR), lambda r: (r, 0, 0)),
            pl.BlockSpec((HID, 128), lambda r: (0, 0)),
            pl.BlockSpec((1, 128), lambda r: (0, 0)),
        ],
        out_specs=pl.BlockSpec((G, 128), lambda r: (0, 0)),
        out_shape=jax.ShapeDtypeStruct((G, 128), jnp.float32),
        scratch_shapes=[
            pltpu.VMEM((G, HID), jnp.float32),
            pltpu.VMEM((G, HID), jnp.float32),
        ],
    )(acc2, deg_t, b2_2d, batch_r, Wc_p, bc_p)


def kernel(x, edge_index, batch, W1, b1, W2, b2, Wc, bc):
    x_p = jnp.pad(x, ((0, NPAD - N), (0, 0)))
    src = jnp.pad(edge_index[0], (0, EP - E))
    dst = jnp.pad(edge_index[1], (0, EP - E), constant_values=N)
    src_r = src.reshape(NS, NCH * CH)
    dst_r = dst.reshape(NS, NCH * CH)
    dst_d = dst.reshape(NC, NS, ED)

    deg_p = _deg_call(dst_d)
    deg_t = deg_p.T

    u1 = _mm1_call(x_p, W1, deg_t)
    acc1 = _agg_call(src_r, dst_r, u1)
    u2 = _mm2_call(acc1, W2, jnp.reshape(b1, (1, HID)), deg_t)
    acc2 = _agg_call(src_r, dst_r, u2)

    batch_r = jnp.pad(batch, (0, NPAD - N), constant_values=G).reshape(NG, 1, TR)
    Wc_p = jnp.pad(Wc, ((0, 0), (0, 128 - C)))
    bc_p = jnp.reshape(jnp.pad(bc, (0, 128 - C)), (1, 128))

    out_p = _final_call(acc2, deg_t, jnp.reshape(b2, (1, HID)), batch_r, Wc_p, bc_p)
    return out_p[:, :C]


# speedup vs baseline: 1.1889x; 1.1161x over previous
import jax
import jax.numpy as jnp
from jax import lax
from jax.experimental import pallas as pl
from jax.experimental.pallas import tpu as pltpu
from jax.experimental.pallas import tpu_sc as plsc

N = 10000
E = 320000
F_IN = 128
HID = 256
C = 3
G = 64

NC = 2
NS = 16

NPAD = 10240
ROWS_PER_TILE = NPAD // NS
CH = 128
IB = 16
NCH = 160
NGRP = NCH // IB
EP = NS * NCH * CH
ED = EP // (NC * NS)
HALF = HID // 2

TR = 2048
NG = NPAD // TR


def _mesh():
    return plsc.VectorSubcoreMesh(
        core_axis_name="c", subcore_axis_name="s", num_cores=NC, num_subcores=NS
    )


def _deg_body(dst_hbm, deg_out, dst_v, ones_v, row_v, deg_sh):
    c = lax.axis_index("c")
    s = lax.axis_index("s")
    pltpu.sync_copy(dst_hbm.at[c, s], dst_v)

    ones16 = jnp.ones((16,), jnp.float32)

    def fill(i, _):
        ones_v[pl.ds(i * 16, 16)] = ones16
        return _

    lax.fori_loop(0, ED // 16, fill, 0)

    zero16 = jnp.zeros((16,), jnp.float32)

    def zfill(i, _):
        row_v[pl.ds(i * 16, 16)] = zero16
        return _

    lax.fori_loop(0, ROWS_PER_TILE // 16, zfill, 0)

    row_lo = s * ROWS_PER_TILE
    pltpu.sync_copy(row_v, deg_sh.at[pl.ds(row_lo, ROWS_PER_TILE)])
    plsc.subcore_barrier()
    pltpu.sync_copy(ones_v, deg_sh.at[dst_v], add=True)
    plsc.subcore_barrier()
    pltpu.sync_copy(deg_sh.at[pl.ds(row_lo, ROWS_PER_TILE)], row_v)
    pltpu.sync_copy(row_v, deg_out.at[c, pl.ds(row_lo, ROWS_PER_TILE)])


def _deg_call(dst_d):
    return pl.kernel(
        _deg_body,
        out_type=jax.ShapeDtypeStruct((NC, NPAD), jnp.float32),
        mesh=_mesh(),
        scratch_types=[
            pltpu.VMEM((ED,), jnp.int32),
            pltpu.VMEM((ED,), jnp.float32),
            pltpu.VMEM((ROWS_PER_TILE,), jnp.float32),
            pltpu.VMEM_SHARED((NPAD,), jnp.float32),
        ],
    )(dst_d)


def _agg_body(src_hbm, dst_hbm, u_hbm, acc_hbm, src_v, dst_v, rows_a, rows_b,
              acc_sh, sem_a, sem_b, sem_sa, sem_sb):
    c = lax.axis_index("c")
    s = lax.axis_index("s")
    row_lo = s * ROWS_PER_TILE

    def run(uc, oc):
        for k in range(ROWS_PER_TILE // CH):
            pltpu.sync_copy(uc.at[pl.ds(row_lo + k * CH, CH)], rows_a)
            pltpu.sync_copy(rows_a, acc_sh.at[pl.ds(row_lo + k * CH, CH)])
        plsc.subcore_barrier()

        def grp(g, _):
            pltpu.sync_copy(src_hbm.at[s, pl.ds(g * IB, IB)], src_v)
            pltpu.sync_copy(dst_hbm.at[s, pl.ds(g * IB, IB)], dst_v)
            pltpu.async_copy(uc.at[src_v.at[0]], rows_a, sem_a)
            pltpu.async_copy(uc.at[src_v.at[1]], rows_b, sem_b)

            def pair(p, _2):
                j0 = 2 * p
                j1 = j0 + 1
                pltpu.make_async_copy(uc.at[src_v.at[j0]], rows_a, sem_a).wait()
                pltpu.async_copy(rows_a, acc_sh.at[dst_v.at[j0]], sem_sa, add=True)
                pltpu.make_async_copy(uc.at[src_v.at[j1]], rows_b, sem_b).wait()
                pltpu.async_copy(rows_b, acc_sh.at[dst_v.at[j1]], sem_sb, add=True)
                pltpu.make_async_copy(rows_a, acc_sh.at[dst_v.at[j0]],
                                      sem_sa).wait()

                @pl.when(j1 + 1 < IB)
                def _prefa():
                    pltpu.async_copy(uc.at[src_v.at[j1 + 1]], rows_a, sem_a)

                pltpu.make_async_copy(rows_b, acc_sh.at[dst_v.at[j1]],
                                      sem_sb).wait()

                @pl.when(j1 + 2 < IB)
                def _prefb():
                    pltpu.async_copy(uc.at[src_v.at[j1 + 2]], rows_b, sem_b)

                return _2

            return lax.fori_loop(0, IB // 2, pair, _)

        lax.fori_loop(0, NGRP, grp, 0)
        plsc.subcore_barrier()
        for k in range(ROWS_PER_TILE // CH):
            pltpu.sync_copy(acc_sh.at[pl.ds(row_lo + k * CH, CH)], rows_a)
            pltpu.sync_copy(rows_a, oc.at[pl.ds(row_lo + k * CH, CH)])

    pl.when(c == 0)(lambda: run(u_hbm.at[0], acc_hbm.at[0]))
    pl.when(c == 1)(lambda: run(u_hbm.at[1], acc_hbm.at[1]))


def _agg_call(src_r, dst_r, u):
    return pl.kernel(
        _agg_body,
        out_type=jax.ShapeDtypeStruct((NC, NPAD, HALF), jnp.float32),
        mesh=_mesh(),
        scratch_types=[
            pltpu.VMEM((IB, CH), jnp.int32),
            pltpu.VMEM((IB, CH), jnp.int32),
            pltpu.VMEM((CH, HALF), jnp.float32),
            pltpu.VMEM((CH, HALF), jnp.float32),
            pltpu.VMEM_SHARED((NPAD, HALF), jnp.float32),
            pltpu.SemaphoreType.DMA,
            pltpu.SemaphoreType.DMA,
            pltpu.SemaphoreType.DMA,
            pltpu.SemaphoreType.DMA,
        ],
    )(src_r, dst_r, u)


def _dinv_of(deg_ref):
    dsum = jnp.sum(deg_ref[...], axis=1, keepdims=True) + 1.0
    return lax.rsqrt(dsum)


def _mm1_body(x_ref, w_ref, deg_ref, u_ref):
    dinv = _dinv_of(deg_ref)
    h = jnp.dot(x_ref[...], w_ref[...], preferred_element_type=jnp.float32)
    u = h * dinv
    u_ref[0] = u[:, :HALF]
    u_ref[1] = u[:, HALF:]


def _mm1_call(x_p, W1, deg_t):
    return pl.pallas_call(
        _mm1_body,
        grid=(NG,),
        in_specs=[
            pl.BlockSpec((TR, F_IN), lambda r: (r, 0)),
            pl.BlockSpec((F_IN, HID), lambda r: (0, 0)),
            pl.BlockSpec((TR, NC), lambda r: (r, 0)),
        ],
        out_specs=pl.BlockSpec((NC, TR, HALF), lambda r: (0, r, 0)),
        out_shape=jax.ShapeDtypeStruct((NC, NPAD, HALF), jnp.float32),
    )(x_p, W1, deg_t)


def _mm2_body(acc_ref, w_ref, b_ref, deg_ref, u_ref):
    dinv = _dinv_of(deg_ref)
    hcat = jnp.concatenate([acc_ref[0], acc_ref[1]], axis=1)
    x2 = jnp.maximum(hcat * dinv + b_ref[...], 0.0)
    h2 = jnp.dot(x2, w_ref[...], preferred_element_type=jnp.float32)
    u = h2 * dinv
    u_ref[0] = u[:, :HALF]
    u_ref[1] = u[:, HALF:]


def _mm2_call(acc1, W2, b1_2d, deg_t):
    return pl.pallas_call(
        _mm2_body,
        grid=(NG,),
        in_specs=[
            pl.BlockSpec((NC, TR, HALF), lambda r: (0, r, 0)),
            pl.BlockSpec((HID, HID), lambda r: (0, 0)),
            pl.BlockSpec((1, HID), lambda r: (0, 0)),
            pl.BlockSpec((TR, NC), lambda r: (r, 0)),
        ],
        out_specs=pl.BlockSpec((NC, TR, HALF), lambda r: (0, r, 0)),
        out_shape=jax.ShapeDtypeStruct((NC, NPAD, HALF), jnp.float32),
    )(acc1, W2, b1_2d, deg_t)


def _final_body(acc_ref, deg_ref, b_ref, batch_ref, wc_ref, bc_ref, out_ref,
                pooled_acc, cnt_acc):
    r = pl.program_id(0)

    @pl.when(r == 0)
    def _init():
        pooled_acc[...] = jnp.zeros((G, HID), jnp.float32)
        cnt_acc[...] = jnp.zeros((G, HID), jnp.float32)

    dinv = _dinv_of(deg_ref)
    hcat = jnp.concatenate([acc_ref[0], acc_ref[1]], axis=1)
    h = hcat * dinv
    b = batch_ref[0]
    oh = (lax.broadcasted_iota(jnp.int32, (G, TR), 0) == b).astype(jnp.float32)
    pooled_acc[...] += jnp.dot(oh, h, preferred_element_type=jnp.float32)
    cnt_acc[...] += jnp.broadcast_to(
        jnp.sum(oh, axis=1, keepdims=True), (G, HID)
    )

    @pl.when(r == NG - 1)
    def _fin():
        cnt = cnt_acc[...]
        sums = pooled_acc[...] + cnt * b_ref[...]
        mean = sums / jnp.maximum(cnt, 1.0)
        out_ref[...] = (
            jnp.dot(mean, wc_ref[...], preferred_element_type=jnp.float32)
            + bc_ref[...]
        )


def _final_call(acc2, deg_t, b2_2d, batch_r, Wc_p, bc_p):
    return pl.pallas_call(
        _final_body,
        grid=(NG,),
        in_specs=[
            pl.BlockSpec((NC, TR, HALF), lambda r: (0, r, 0)),
            pl.BlockSpec((TR, NC), lambda r: (r, 0)),
            pl.BlockSpec((1, HID), lambda r: (0, 0)),
            pl.BlockSpec((1, 1, TR), lambda r: (r, 0, 0)),
            pl.BlockSpec((HID, 128), lambda r: (0, 0)),
            pl.BlockSpec((1, 128), lambda r: (0, 0)),
        ],
        out_specs=pl.BlockSpec((G, 128), lambda r: (0, 0)),
        out_shape=jax.ShapeDtypeStruct((G, 128), jnp.float32),
        scratch_shapes=[
            pltpu.VMEM((G, HID), jnp.float32),
            pltpu.VMEM((G, HID), jnp.float32),
        ],
    )(acc2, deg_t, b2_2d, batch_r, Wc_p, bc_p)


def kernel(x, edge_index, batch, W1, b1, W2, b2, Wc, bc):
    x_p = jnp.pad(x, ((0, NPAD - N), (0, 0)))
    src = jnp.pad(edge_index[0], (0, EP - E))
    dst = jnp.pad(edge_index[1], (0, EP - E), constant_values=N)
    src_r = src.reshape(NS, NCH, CH)
    dst_r = dst.reshape(NS, NCH, CH)
    dst_d = dst.reshape(NC, NS, ED)

    deg_p = _deg_call(dst_d)
    deg_t = deg_p.T

    u1 = _mm1_call(x_p, W1, deg_t)
    acc1 = _agg_call(src_r, dst_r, u1)
    u2 = _mm2_call(acc1, W2, jnp.reshape(b1, (1, HID)), deg_t)
    acc2 = _agg_call(src_r, dst_r, u2)

    batch_r = jnp.pad(batch, (0, NPAD - N), constant_values=G).reshape(NG, 1, TR)
    Wc_p = jnp.pad(Wc, ((0, 0), (0, 128 - C)))
    bc_p = jnp.reshape(jnp.pad(bc, (0, 128 - C)), (1, 128))

    out_p = _final_call(acc2, deg_t, jnp.reshape(b2, (1, HID)), batch_r, Wc_p, bc_p)
    return out_p[:, :C]


# speedup vs baseline: 1.2809x; 1.0774x over previous
import jax
import jax.numpy as jnp
from jax import lax
from jax.experimental import pallas as pl
from jax.experimental.pallas import tpu as pltpu
from jax.experimental.pallas import tpu_sc as plsc

N = 10000
E = 320000
F_IN = 128
HID = 256
C = 3
G = 64

NC = 2
NS = 16

NPAD = 10240
ROWS_PER_TILE = NPAD // NS
CH = 128
IB = 16
NCH = 160
NGRP = NCH // IB
EP = NS * NCH * CH
ED = EP // (NC * NS)
HALF = HID // 2

TR = 2048
NG = NPAD // TR


def _mesh():
    return plsc.VectorSubcoreMesh(
        core_axis_name="c", subcore_axis_name="s", num_cores=NC, num_subcores=NS
    )


def _deg_body(dst_hbm, deg_out, dst_v, ones_v, row_v, deg_sh):
    c = lax.axis_index("c")
    s = lax.axis_index("s")
    pltpu.sync_copy(dst_hbm.at[c, s], dst_v)

    ones16 = jnp.ones((16,), jnp.float32)

    def fill(i, _):
        ones_v[pl.ds(i * 16, 16)] = ones16
        return _

    lax.fori_loop(0, ED // 16, fill, 0)

    zero16 = jnp.zeros((16,), jnp.float32)

    def zfill(i, _):
        row_v[pl.ds(i * 16, 16)] = zero16
        return _

    lax.fori_loop(0, ROWS_PER_TILE // 16, zfill, 0)

    row_lo = s * ROWS_PER_TILE
    pltpu.sync_copy(row_v, deg_sh.at[pl.ds(row_lo, ROWS_PER_TILE)])
    plsc.subcore_barrier()
    pltpu.sync_copy(ones_v, deg_sh.at[dst_v], add=True)
    plsc.subcore_barrier()
    pltpu.sync_copy(deg_sh.at[pl.ds(row_lo, ROWS_PER_TILE)], row_v)
    pltpu.sync_copy(row_v, deg_out.at[c, pl.ds(row_lo, ROWS_PER_TILE)])


def _deg_call(dst_d):
    return pl.kernel(
        _deg_body,
        out_type=jax.ShapeDtypeStruct((NC, NPAD), jnp.float32),
        mesh=_mesh(),
        scratch_types=[
            pltpu.VMEM((ED,), jnp.int32),
            pltpu.VMEM((ED,), jnp.float32),
            pltpu.VMEM((ROWS_PER_TILE,), jnp.float32),
            pltpu.VMEM_SHARED((NPAD,), jnp.float32),
        ],
    )(dst_d)


def _agg_body(src_hbm, dst_hbm, u_hbm, acc_hbm, src_v, dst_v, rows_a, rows_b,
              acc_sh, sem_a, sem_b):
    c = lax.axis_index("c")
    s = lax.axis_index("s")
    row_lo = s * ROWS_PER_TILE

    def run(uc, oc):
        pltpu.sync_copy(uc.at[pl.ds(row_lo, ROWS_PER_TILE)],
                        acc_sh.at[pl.ds(row_lo, ROWS_PER_TILE)])
        plsc.subcore_barrier()

        def grp(g, _):
            pltpu.sync_copy(src_hbm.at[s, pl.ds(g * IB, IB)], src_v)
            pltpu.sync_copy(dst_hbm.at[s, pl.ds(g * IB, IB)], dst_v)
            pltpu.async_copy(uc.at[src_v.at[0]], rows_a, sem_a)

            def pair(p, _2):
                j0 = 2 * p
                j1 = j0 + 1
                pltpu.make_async_copy(uc.at[src_v.at[j0]], rows_a, sem_a).wait()
                pltpu.async_copy(uc.at[src_v.at[j1]], rows_b, sem_b)
                pltpu.sync_copy(rows_a, acc_sh.at[dst_v.at[j0]], add=True)

                @pl.when(j1 + 1 < IB)
                def _pref():
                    pltpu.async_copy(uc.at[src_v.at[j1 + 1]], rows_a, sem_a)

                pltpu.make_async_copy(uc.at[src_v.at[j1]], rows_b, sem_b).wait()
                pltpu.sync_copy(rows_b, acc_sh.at[dst_v.at[j1]], add=True)
                return _2

            return lax.fori_loop(0, IB // 2, pair, _)

        lax.fori_loop(0, NGRP, grp, 0)
        plsc.subcore_barrier()
        pltpu.sync_copy(acc_sh.at[pl.ds(row_lo, ROWS_PER_TILE)],
                        oc.at[pl.ds(row_lo, ROWS_PER_TILE)])

    pl.when(c == 0)(lambda: run(u_hbm.at[0], acc_hbm.at[0]))
    pl.when(c == 1)(lambda: run(u_hbm.at[1], acc_hbm.at[1]))


def _agg_call(src_r, dst_r, u):
    return pl.kernel(
        _agg_body,
        out_type=jax.ShapeDtypeStruct((NC, NPAD, HALF), jnp.float32),
        mesh=_mesh(),
        scratch_types=[
            pltpu.VMEM((IB, CH), jnp.int32),
            pltpu.VMEM((IB, CH), jnp.int32),
            pltpu.VMEM((CH, HALF), jnp.float32),
            pltpu.VMEM((CH, HALF), jnp.float32),
            pltpu.VMEM_SHARED((NPAD, HALF), jnp.float32),
            pltpu.SemaphoreType.DMA,
            pltpu.SemaphoreType.DMA,
        ],
    )(src_r, dst_r, u)


def _dinv_of(deg_ref):
    dsum = jnp.sum(deg_ref[...], axis=1, keepdims=True) + 1.0
    return lax.rsqrt(dsum)


def _mm1_body(x_ref, w_ref, deg_ref, u_ref):
    dinv = _dinv_of(deg_ref)
    h = jnp.dot(x_ref[...], w_ref[...], preferred_element_type=jnp.float32)
    u = h * dinv
    u_ref[0] = u[:, :HALF]
    u_ref[1] = u[:, HALF:]


def _mm1_call(x_p, W1, deg_t):
    return pl.pallas_call(
        _mm1_body,
        grid=(NG,),
        in_specs=[
            pl.BlockSpec((TR, F_IN), lambda r: (r, 0)),
            pl.BlockSpec((F_IN, HID), lambda r: (0, 0)),
            pl.BlockSpec((TR, NC), lambda r: (r, 0)),
        ],
        out_specs=pl.BlockSpec((NC, TR, HALF), lambda r: (0, r, 0)),
        out_shape=jax.ShapeDtypeStruct((NC, NPAD, HALF), jnp.float32),
    )(x_p, W1, deg_t)


def _mm2_body(acc_ref, w_ref, b_ref, deg_ref, u_ref):
    dinv = _dinv_of(deg_ref)
    hcat = jnp.concatenate([acc_ref[0], acc_ref[1]], axis=1)
    x2 = jnp.maximum(hcat * dinv + b_ref[...], 0.0)
    h2 = jnp.dot(x2, w_ref[...], preferred_element_type=jnp.float32)
    u = h2 * dinv
    u_ref[0] = u[:, :HALF]
    u_ref[1] = u[:, HALF:]


def _mm2_call(acc1, W2, b1_2d, deg_t):
    return pl.pallas_call(
        _mm2_body,
        grid=(NG,),
        in_specs=[
            pl.BlockSpec((NC, TR, HALF), lambda r: (0, r, 0)),
            pl.BlockSpec((HID, HID), lambda r: (0, 0)),
            pl.BlockSpec((1, HID), lambda r: (0, 0)),
            pl.BlockSpec((TR, NC), lambda r: (r, 0)),
        ],
        out_specs=pl.BlockSpec((NC, TR, HALF), lambda r: (0, r, 0)),
        out_shape=jax.ShapeDtypeStruct((NC, NPAD, HALF), jnp.float32),
    )(acc1, W2, b1_2d, deg_t)


def _final_body(acc_ref, deg_ref, b_ref, batch_ref, wc_ref, bc_ref, out_ref,
                pooled_acc, cnt_acc):
    r = pl.program_id(0)

    @pl.when(r == 0)
    def _init():
        pooled_acc[...] = jnp.zeros((G, HID), jnp.float32)
        cnt_acc[...] = jnp.zeros((G, HID), jnp.float32)

    dinv = _dinv_of(deg_ref)
    hcat = jnp.concatenate([acc_ref[0], acc_ref[1]], axis=1)
    h = hcat * dinv
    b = batch_ref[0]
    oh = (lax.broadcasted_iota(jnp.int32, (G, TR), 0) == b).astype(jnp.float32)
    pooled_acc[...] += jnp.dot(oh, h, preferred_element_type=jnp.float32)
    cnt_acc[...] += jnp.broadcast_to(
        jnp.sum(oh, axis=1, keepdims=True), (G, HID)
    )

    @pl.when(r == NG - 1)
    def _fin():
        cnt = cnt_acc[...]
        sums = pooled_acc[...] + cnt * b_ref[...]
        mean = sums / jnp.maximum(cnt, 1.0)
        out_ref[...] = (
            jnp.dot(mean, wc_ref[...], preferred_element_type=jnp.float32)
            + bc_ref[...]
        )


def _final_call(acc2, deg_t, b2_2d, batch_r, Wc_p, bc_p):
    return pl.pallas_call(
        _final_body,
        grid=(NG,),
        in_specs=[
            pl.BlockSpec((NC, TR, HALF), lambda r: (0, r, 0)),
            pl.BlockSpec((TR, NC), lambda r: (r, 0)),
            pl.BlockSpec((1, HID), lambda r: (0, 0)),
            pl.BlockSpec((1, 1, TR), lambda r: (r, 0, 0)),
            pl.BlockSpec((HID, 128), lambda r: (0, 0)),
            pl.BlockSpec((1, 128), lambda r: (0, 0)),
        ],
        out_specs=pl.BlockSpec((G, 128), lambda r: (0, 0)),
        out_shape=jax.ShapeDtypeStruct((G, 128), jnp.float32),
        scratch_shapes=[
            pltpu.VMEM((G, HID), jnp.float32),
            pltpu.VMEM((G, HID), jnp.float32),
        ],
    )(acc2, deg_t, b2_2d, batch_r, Wc_p, bc_p)


def kernel(x, edge_index, batch, W1, b1, W2, b2, Wc, bc):
    x_p = jnp.pad(x, ((0, NPAD - N), (0, 0)))
    src = jnp.pad(edge_index[0], (0, EP - E))
    dst = jnp.pad(edge_index[1], (0, EP - E), constant_values=N)
    src_r = src.reshape(NS, NCH, CH)
    dst_r = dst.reshape(NS, NCH, CH)
    dst_d = dst.reshape(NC, NS, ED)

    deg_p = _deg_call(dst_d)
    deg_t = deg_p.T

    u1 = _mm1_call(x_p, W1, deg_t)
    acc1 = _agg_call(src_r, dst_r, u1)
    u2 = _mm2_call(acc1, W2, jnp.reshape(b1, (1, HID)), deg_t)
    acc2 = _agg_call(src_r, dst_r, u2)

    batch_r = jnp.pad(batch, (0, NPAD - N), constant_values=G).reshape(NG, 1, TR)
    Wc_p = jnp.pad(Wc, ((0, 0), (0, 128 - C)))
    bc_p = jnp.reshape(jnp.pad(bc, (0, 128 - C)), (1, 128))

    out_p = _final_call(acc2, deg_t, jnp.reshape(b2, (1, HID)), batch_r, Wc_p, bc_p)
    return out_p[:, :C]


# speedup vs baseline: 1.3155x; 1.0270x over previous
import jax
import jax.numpy as jnp
from jax import lax
from jax.experimental import pallas as pl
from jax.experimental.pallas import tpu as pltpu
from jax.experimental.pallas import tpu_sc as plsc

N = 10000
E = 320000
F_IN = 128
HID = 256
C = 3
G = 64

NC = 2
NS = 16

NPAD = 10240
ROWS_PER_TILE = NPAD // NS
CH = 128
IB = 8
NCH = 160
NGRP = NCH // IB
EP = NS * NCH * CH
ED = EP // (NC * NS)
HALF = HID // 2

TR = 2048
NG = NPAD // TR


def _mesh():
    return plsc.VectorSubcoreMesh(
        core_axis_name="c", subcore_axis_name="s", num_cores=NC, num_subcores=NS
    )


def _deg_body(dst_hbm, deg_out, dst_v, ones_v, row_v, deg_sh):
    c = lax.axis_index("c")
    s = lax.axis_index("s")
    pltpu.sync_copy(dst_hbm.at[c, s], dst_v)

    ones16 = jnp.ones((16,), jnp.float32)

    def fill(i, _):
        ones_v[pl.ds(i * 16, 16)] = ones16
        return _

    lax.fori_loop(0, ED // 16, fill, 0)

    zero16 = jnp.zeros((16,), jnp.float32)

    def zfill(i, _):
        row_v[pl.ds(i * 16, 16)] = zero16
        return _

    lax.fori_loop(0, ROWS_PER_TILE // 16, zfill, 0)

    row_lo = s * ROWS_PER_TILE
    pltpu.sync_copy(row_v, deg_sh.at[pl.ds(row_lo, ROWS_PER_TILE)])
    plsc.subcore_barrier()
    pltpu.sync_copy(ones_v, deg_sh.at[dst_v], add=True)
    plsc.subcore_barrier()
    pltpu.sync_copy(deg_sh.at[pl.ds(row_lo, ROWS_PER_TILE)], row_v)
    pltpu.sync_copy(row_v, deg_out.at[c, pl.ds(row_lo, ROWS_PER_TILE)])


def _deg_call(dst_d):
    return pl.kernel(
        _deg_body,
        out_type=jax.ShapeDtypeStruct((NC, NPAD), jnp.float32),
        mesh=_mesh(),
        scratch_types=[
            pltpu.VMEM((ED,), jnp.int32),
            pltpu.VMEM((ED,), jnp.float32),
            pltpu.VMEM((ROWS_PER_TILE,), jnp.float32),
            pltpu.VMEM_SHARED((NPAD,), jnp.float32),
        ],
    )(dst_d)


def _agg_body(src_hbm, dst_hbm, u_hbm, acc_hbm, src_0, src_1, dst_0, dst_1,
              rows_a, rows_b, acc_sh, sem_a, sem_b, sem_i, sem_j):
    src_i = (src_0, src_1)
    dst_i = (dst_0, dst_1)
    c = lax.axis_index("c")
    s = lax.axis_index("s")
    row_lo = s * ROWS_PER_TILE

    def run(uc, oc):
        pltpu.sync_copy(uc.at[pl.ds(row_lo, ROWS_PER_TILE)],
                        acc_sh.at[pl.ds(row_lo, ROWS_PER_TILE)])
        plsc.subcore_barrier()

        pltpu.sync_copy(src_hbm.at[s, pl.ds(0, IB)], src_i[0])
        pltpu.sync_copy(dst_hbm.at[s, pl.ds(0, IB)], dst_i[0])
        pltpu.async_copy(uc.at[src_i[0].at[0]], rows_a, sem_a)

        def grp(g, _):
            def half(q):
                src_v = src_i[q]
                dst_v = dst_i[q]
                nsrc_v = src_i[1 - q]

                @pl.when(g + 1 < NGRP)
                def _ldnext():
                    pltpu.async_copy(
                        src_hbm.at[s, pl.ds((g + 1) * IB, IB)], src_i[1 - q],
                        sem_i)
                    pltpu.async_copy(
                        dst_hbm.at[s, pl.ds((g + 1) * IB, IB)], dst_i[1 - q],
                        sem_j)

                def pair(p, _2):
                    j0 = 2 * p
                    j1 = j0 + 1
                    pltpu.make_async_copy(
                        uc.at[src_v.at[j0]], rows_a, sem_a).wait()
                    pltpu.async_copy(uc.at[src_v.at[j1]], rows_b, sem_b)
                    pltpu.sync_copy(rows_a, acc_sh.at[dst_v.at[j0]], add=True)

                    @pl.when(j1 + 1 < IB)
                    def _pref():
                        pltpu.async_copy(uc.at[src_v.at[j1 + 1]], rows_a, sem_a)

                    pltpu.make_async_copy(
                        uc.at[src_v.at[j1]], rows_b, sem_b).wait()
                    pltpu.sync_copy(rows_b, acc_sh.at[dst_v.at[j1]], add=True)
                    return _2

                lax.fori_loop(0, IB // 2 - 1, pair, 0)
                j0 = IB - 2
                j1 = IB - 1
                pltpu.make_async_copy(uc.at[src_v.at[j0]], rows_a, sem_a).wait()
                pltpu.async_copy(uc.at[src_v.at[j1]], rows_b, sem_b)
                pltpu.sync_copy(rows_a, acc_sh.at[dst_v.at[j0]], add=True)

                @pl.when(g + 1 < NGRP)
                def _prefn():
                    pltpu.make_async_copy(
                        src_hbm.at[s, pl.ds((g + 1) * IB, IB)], nsrc_v,
                        sem_i).wait()
                    pltpu.async_copy(uc.at[nsrc_v.at[0]], rows_a, sem_a)
                    pltpu.make_async_copy(
                        dst_hbm.at[s, pl.ds((g + 1) * IB, IB)], dst_i[1 - q],
                        sem_j).wait()

                pltpu.make_async_copy(uc.at[src_v.at[j1]], rows_b, sem_b).wait()
                pltpu.sync_copy(rows_b, acc_sh.at[dst_v.at[j1]], add=True)

            pl.when(g % 2 == 0)(lambda: half(0))
            pl.when(g % 2 == 1)(lambda: half(1))
            return _

        lax.fori_loop(0, NGRP, grp, 0)
        plsc.subcore_barrier()
        pltpu.sync_copy(acc_sh.at[pl.ds(row_lo, ROWS_PER_TILE)],
                        oc.at[pl.ds(row_lo, ROWS_PER_TILE)])

    pl.when(c == 0)(lambda: run(u_hbm.at[0], acc_hbm.at[0]))
    pl.when(c == 1)(lambda: run(u_hbm.at[1], acc_hbm.at[1]))


def _agg_call(src_r, dst_r, u):
    return pl.kernel(
        _agg_body,
        out_type=jax.ShapeDtypeStruct((NC, NPAD, HALF), jnp.float32),
        mesh=_mesh(),
        scratch_types=[
            pltpu.VMEM((IB, CH), jnp.int32),
            pltpu.VMEM((IB, CH), jnp.int32),
            pltpu.VMEM((IB, CH), jnp.int32),
            pltpu.VMEM((IB, CH), jnp.int32),
            pltpu.VMEM((CH, HALF), jnp.float32),
            pltpu.VMEM((CH, HALF), jnp.float32),
            pltpu.VMEM_SHARED((NPAD, HALF), jnp.float32),
            pltpu.SemaphoreType.DMA,
            pltpu.SemaphoreType.DMA,
            pltpu.SemaphoreType.DMA,
            pltpu.SemaphoreType.DMA,
        ],
    )(src_r, dst_r, u)


def _dinv_of(deg_ref):
    dsum = jnp.sum(deg_ref[...], axis=1, keepdims=True) + 1.0
    return lax.rsqrt(dsum)


def _mm1_body(x_ref, w_ref, deg_ref, u_ref):
    dinv = _dinv_of(deg_ref)
    h = jnp.dot(x_ref[...], w_ref[...], preferred_element_type=jnp.float32)
    u = h * dinv
    u_ref[0] = u[:, :HALF]
    u_ref[1] = u[:, HALF:]


def _mm1_call(x_p, W1, deg_t):
    return pl.pallas_call(
        _mm1_body,
        grid=(NG,),
        in_specs=[
            pl.BlockSpec((TR, F_IN), lambda r: (r, 0)),
            pl.BlockSpec((F_IN, HID), lambda r: (0, 0)),
            pl.BlockSpec((TR, NC), lambda r: (r, 0)),
        ],
        out_specs=pl.BlockSpec((NC, TR, HALF), lambda r: (0, r, 0)),
        out_shape=jax.ShapeDtypeStruct((NC, NPAD, HALF), jnp.float32),
    )(x_p, W1, deg_t)


def _mm2_body(acc_ref, w_ref, b_ref, deg_ref, u_ref):
    dinv = _dinv_of(deg_ref)
    hcat = jnp.concatenate([acc_ref[0], acc_ref[1]], axis=1)
    x2 = jnp.maximum(hcat * dinv + b_ref[...], 0.0)
    h2 = jnp.dot(x2, w_ref[...], preferred_element_type=jnp.float32)
    u = h2 * dinv
    u_ref[0] = u[:, :HALF]
    u_ref[1] = u[:, HALF:]


def _mm2_call(acc1, W2, b1_2d, deg_t):
    return pl.pallas_call(
        _mm2_body,
        grid=(NG,),
        in_specs=[
            pl.BlockSpec((NC, TR, HALF), lambda r: (0, r, 0)),
            pl.BlockSpec((HID, HID), lambda r: (0, 0)),
            pl.BlockSpec((1, HID), lambda r: (0, 0)),
            pl.BlockSpec((TR, NC), lambda r: (r, 0)),
        ],
        out_specs=pl.BlockSpec((NC, TR, HALF), lambda r: (0, r, 0)),
        out_shape=jax.ShapeDtypeStruct((NC, NPAD, HALF), jnp.float32),
    )(acc1, W2, b1_2d, deg_t)


def _final_body(acc_ref, deg_ref, b_ref, batch_ref, wc_ref, bc_ref, out_ref,
                pooled_acc, cnt_acc):
    r = pl.program_id(0)

    @pl.when(r == 0)
    def _init():
        pooled_acc[...] = jnp.zeros((G, HID), jnp.float32)
        cnt_acc[...] = jnp.zeros((G, HID), jnp.float32)

    dinv = _dinv_of(deg_ref)
    hcat = jnp.concatenate([acc_ref[0], acc_ref[1]], axis=1)
    h = hcat * dinv
    b = batch_ref[0]
    oh = (lax.broadcasted_iota(jnp.int32, (G, TR), 0) == b).astype(jnp.float32)
    pooled_acc[...] += jnp.dot(oh, h, preferred_element_type=jnp.float32)
    cnt_acc[...] += jnp.broadcast_to(
        jnp.sum(oh, axis=1, keepdims=True), (G, HID)
    )

    @pl.when(r == NG - 1)
    def _fin():
        cnt = cnt_acc[...]
        sums = pooled_acc[...] + cnt * b_ref[...]
        mean = sums / jnp.maximum(cnt, 1.0)
        out_ref[...] = (
            jnp.dot(mean, wc_ref[...], preferred_element_type=jnp.float32)
            + bc_ref[...]
        )


def _final_call(acc2, deg_t, b2_2d, batch_r, Wc_p, bc_p):
    return pl.pallas_call(
        _final_body,
        grid=(NG,),
        in_specs=[
            pl.BlockSpec((NC, TR, HALF), lambda r: (0, r, 0)),
            pl.BlockSpec((TR, NC), lambda r: (r, 0)),
            pl.BlockSpec((1, HID), lambda r: (0, 0)),
            pl.BlockSpec((1, 1, TR), lambda r: (r, 0, 0)),
            pl.BlockSpec((HID, 128), lambda r: (0, 0)),
            pl.BlockSpec((1, 128), lambda r: (0, 0)),
        ],
        out_specs=pl.BlockSpec((G, 128), lambda r: (0, 0)),
        out_shape=jax.ShapeDtypeStruct((G, 128), jnp.float32),
        scratch_shapes=[
            pltpu.VMEM((G, HID), jnp.float32),
            pltpu.VMEM((G, HID), jnp.float32),
        ],
    )(acc2, deg_t, b2_2d, batch_r, Wc_p, bc_p)


def kernel(x, edge_index, batch, W1, b1, W2, b2, Wc, bc):
    x_p = jnp.pad(x, ((0, NPAD - N), (0, 0)))
    src = jnp.pad(edge_index[0], (0, EP - E))
    dst = jnp.pad(edge_index[1], (0, EP - E), constant_values=N)
    src_r = src.reshape(NS, NCH, CH)
    dst_r = dst.reshape(NS, NCH, CH)
    dst_d = dst.reshape(NC, NS, ED)

    deg_p = _deg_call(dst_d)
    deg_t = deg_p.T

    u1 = _mm1_call(x_p, W1, deg_t)
    acc1 = _agg_call(src_r, dst_r, u1)
    u2 = _mm2_call(acc1, W2, jnp.reshape(b1, (1, HID)), deg_t)
    acc2 = _agg_call(src_r, dst_r, u2)

    batch_r = jnp.pad(batch, (0, NPAD - N), constant_values=G).reshape(NG, 1, TR)
    Wc_p = jnp.pad(Wc, ((0, 0), (0, 128 - C)))
    bc_p = jnp.reshape(jnp.pad(bc, (0, 128 - C)), (1, 128))

    out_p = _final_call(acc2, deg_t, jnp.reshape(b2, (1, HID)), batch_r, Wc_p, bc_p)
    return out_p[:, :C]


# speedup vs baseline: 1.3222x; 1.0051x over previous
import jax
import jax.numpy as jnp
from jax import lax
from jax.experimental import pallas as pl
from jax.experimental.pallas import tpu as pltpu
from jax.experimental.pallas import tpu_sc as plsc

N = 10000
E = 320000
F_IN = 128
HID = 256
C = 3
G = 64

NC = 2
NS = 16

NPAD = 10240
ROWS_PER_TILE = NPAD // NS
CH = 128
IB = 8
NCH = 160
NGRP = NCH // IB
EP = NS * NCH * CH
ED = EP // (NC * NS)
HALF = HID // 2

TR = 2048
NG = NPAD // TR


def _mesh():
    return plsc.VectorSubcoreMesh(
        core_axis_name="c", subcore_axis_name="s", num_cores=NC, num_subcores=NS
    )


def _deg_body(dst_hbm, deg_out, dst_v, ones_v, row_v, deg_sh):
    c = lax.axis_index("c")
    s = lax.axis_index("s")
    pltpu.sync_copy(dst_hbm.at[c, s], dst_v)

    ones16 = jnp.ones((16,), jnp.float32)

    def fill(i, _):
        ones_v[pl.ds(i * 16, 16)] = ones16
        return _

    lax.fori_loop(0, ED // 16, fill, 0)

    zero16 = jnp.zeros((16,), jnp.float32)

    def zfill(i, _):
        row_v[pl.ds(i * 16, 16)] = zero16
        return _

    lax.fori_loop(0, ROWS_PER_TILE // 16, zfill, 0)

    row_lo = s * ROWS_PER_TILE
    pltpu.sync_copy(row_v, deg_sh.at[pl.ds(row_lo, ROWS_PER_TILE)])
    plsc.subcore_barrier()
    pltpu.sync_copy(ones_v, deg_sh.at[dst_v], add=True)
    plsc.subcore_barrier()
    pltpu.sync_copy(deg_sh.at[pl.ds(row_lo, ROWS_PER_TILE)], row_v)
    pltpu.sync_copy(row_v, deg_out.at[c, pl.ds(row_lo, ROWS_PER_TILE)])


def _deg_call(dst_d):
    return pl.kernel(
        _deg_body,
        out_type=jax.ShapeDtypeStruct((NC, NPAD), jnp.float32),
        mesh=_mesh(),
        scratch_types=[
            pltpu.VMEM((ED,), jnp.int32),
            pltpu.VMEM((ED,), jnp.float32),
            pltpu.VMEM((ROWS_PER_TILE,), jnp.float32),
            pltpu.VMEM_SHARED((NPAD,), jnp.float32),
        ],
    )(dst_d)


def _agg_body(src_hbm, dst_hbm, u_hbm, acc_hbm, src_0, src_1, dst_0, dst_1,
              rows_a, rows_b, acc_sh, sem_a, sem_b, sem_i, sem_j):
    src_i = (src_0, src_1)
    dst_i = (dst_0, dst_1)
    c = lax.axis_index("c")
    s = lax.axis_index("s")
    row_lo = s * ROWS_PER_TILE

    def run(uc, oc):
        pltpu.async_copy(uc.at[pl.ds(row_lo, ROWS_PER_TILE)],
                         acc_sh.at[pl.ds(row_lo, ROWS_PER_TILE)], sem_i)
        pltpu.sync_copy(src_hbm.at[s, pl.ds(0, IB)], src_i[0])
        pltpu.sync_copy(dst_hbm.at[s, pl.ds(0, IB)], dst_i[0])
        pltpu.async_copy(uc.at[src_i[0].at[0]], rows_a, sem_a)
        pltpu.make_async_copy(uc.at[pl.ds(row_lo, ROWS_PER_TILE)],
                              acc_sh.at[pl.ds(row_lo, ROWS_PER_TILE)],
                              sem_i).wait()
        plsc.subcore_barrier()

        def grp(g, _):
            def half(q):
                src_v = src_i[q]
                dst_v = dst_i[q]
                nsrc_v = src_i[1 - q]

                @pl.when(g + 1 < NGRP)
                def _ldnext():
                    pltpu.async_copy(
                        src_hbm.at[s, pl.ds((g + 1) * IB, IB)], src_i[1 - q],
                        sem_i)
                    pltpu.async_copy(
                        dst_hbm.at[s, pl.ds((g + 1) * IB, IB)], dst_i[1 - q],
                        sem_j)

                def pair(p, _2):
                    j0 = 2 * p
                    j1 = j0 + 1
                    pltpu.make_async_copy(
                        uc.at[src_v.at[j0]], rows_a, sem_a).wait()
                    pltpu.async_copy(uc.at[src_v.at[j1]], rows_b, sem_b)
                    pltpu.sync_copy(rows_a, acc_sh.at[dst_v.at[j0]], add=True)

                    @pl.when(j1 + 1 < IB)
                    def _pref():
                        pltpu.async_copy(uc.at[src_v.at[j1 + 1]], rows_a, sem_a)

                    pltpu.make_async_copy(
                        uc.at[src_v.at[j1]], rows_b, sem_b).wait()
                    pltpu.sync_copy(rows_b, acc_sh.at[dst_v.at[j1]], add=True)
                    return _2

                lax.fori_loop(0, IB // 2 - 1, pair, 0)
                j0 = IB - 2
                j1 = IB - 1
                pltpu.make_async_copy(uc.at[src_v.at[j0]], rows_a, sem_a).wait()
                pltpu.async_copy(uc.at[src_v.at[j1]], rows_b, sem_b)
                pltpu.sync_copy(rows_a, acc_sh.at[dst_v.at[j0]], add=True)

                @pl.when(g + 1 < NGRP)
                def _prefn():
                    pltpu.make_async_copy(
                        src_hbm.at[s, pl.ds((g + 1) * IB, IB)], nsrc_v,
                        sem_i).wait()
                    pltpu.async_copy(uc.at[nsrc_v.at[0]], rows_a, sem_a)
                    pltpu.make_async_copy(
                        dst_hbm.at[s, pl.ds((g + 1) * IB, IB)], dst_i[1 - q],
                        sem_j).wait()

                pltpu.make_async_copy(uc.at[src_v.at[j1]], rows_b, sem_b).wait()
                pltpu.sync_copy(rows_b, acc_sh.at[dst_v.at[j1]], add=True)

            pl.when(g % 2 == 0)(lambda: half(0))
            pl.when(g % 2 == 1)(lambda: half(1))
            return _

        lax.fori_loop(0, NGRP, grp, 0)
        plsc.subcore_barrier()
        pltpu.sync_copy(acc_sh.at[pl.ds(row_lo, ROWS_PER_TILE)],
                        oc.at[pl.ds(row_lo, ROWS_PER_TILE)])

    pl.when(c == 0)(lambda: run(u_hbm.at[0], acc_hbm.at[0]))
    pl.when(c == 1)(lambda: run(u_hbm.at[1], acc_hbm.at[1]))


def _agg_call(src_r, dst_r, u):
    return pl.kernel(
        _agg_body,
        out_type=jax.ShapeDtypeStruct((NC, NPAD, HALF), jnp.float32),
        mesh=_mesh(),
        scratch_types=[
            pltpu.VMEM((IB, CH), jnp.int32),
            pltpu.VMEM((IB, CH), jnp.int32),
            pltpu.VMEM((IB, CH), jnp.int32),
            pltpu.VMEM((IB, CH), jnp.int32),
            pltpu.VMEM((CH, HALF), jnp.float32),
            pltpu.VMEM((CH, HALF), jnp.float32),
            pltpu.VMEM_SHARED((NPAD, HALF), jnp.float32),
            pltpu.SemaphoreType.DMA,
            pltpu.SemaphoreType.DMA,
            pltpu.SemaphoreType.DMA,
            pltpu.SemaphoreType.DMA,
        ],
    )(src_r, dst_r, u)


def _dinv_of(deg_ref):
    dsum = jnp.sum(deg_ref[...], axis=1, keepdims=True) + 1.0
    return lax.rsqrt(dsum)


def _mm1_body(x_ref, w_ref, deg_ref, u_ref):
    dinv = _dinv_of(deg_ref)
    h = jnp.dot(x_ref[...], w_ref[...], preferred_element_type=jnp.float32)
    u = h * dinv
    u_ref[0] = u[:, :HALF]
    u_ref[1] = u[:, HALF:]


def _mm1_call(x_p, W1, deg_t):
    return pl.pallas_call(
        _mm1_body,
        grid=(NG,),
        in_specs=[
            pl.BlockSpec((TR, F_IN), lambda r: (r, 0)),
            pl.BlockSpec((F_IN, HID), lambda r: (0, 0)),
            pl.BlockSpec((TR, NC), lambda r: (r, 0)),
        ],
        out_specs=pl.BlockSpec((NC, TR, HALF), lambda r: (0, r, 0)),
        out_shape=jax.ShapeDtypeStruct((NC, NPAD, HALF), jnp.float32),
    )(x_p, W1, deg_t)


def _mm2_body(acc_ref, w_ref, b_ref, deg_ref, u_ref):
    dinv = _dinv_of(deg_ref)
    hcat = jnp.concatenate([acc_ref[0], acc_ref[1]], axis=1)
    x2 = jnp.maximum(hcat * dinv + b_ref[...], 0.0)
    h2 = jnp.dot(x2, w_ref[...], preferred_element_type=jnp.float32)
    u = h2 * dinv
    u_ref[0] = u[:, :HALF]
    u_ref[1] = u[:, HALF:]


def _mm2_call(acc1, W2, b1_2d, deg_t):
    return pl.pallas_call(
        _mm2_body,
        grid=(NG,),
        in_specs=[
            pl.BlockSpec((NC, TR, HALF), lambda r: (0, r, 0)),
            pl.BlockSpec((HID, HID), lambda r: (0, 0)),
            pl.BlockSpec((1, HID), lambda r: (0, 0)),
            pl.BlockSpec((TR, NC), lambda r: (r, 0)),
        ],
        out_specs=pl.BlockSpec((NC, TR, HALF), lambda r: (0, r, 0)),
        out_shape=jax.ShapeDtypeStruct((NC, NPAD, HALF), jnp.float32),
    )(acc1, W2, b1_2d, deg_t)


def _final_body(acc_ref, deg_ref, b_ref, batch_ref, wc_ref, bc_ref, out_ref,
                pooled_acc, cnt_acc):
    r = pl.program_id(0)

    @pl.when(r == 0)
    def _init():
        pooled_acc[...] = jnp.zeros((G, HID), jnp.float32)
        cnt_acc[...] = jnp.zeros((G, HID), jnp.float32)

    dinv = _dinv_of(deg_ref)
    hcat = jnp.concatenate([acc_ref[0], acc_ref[1]], axis=1)
    h = hcat * dinv
    b = batch_ref[0]
    oh = (lax.broadcasted_iota(jnp.int32, (G, TR), 0) == b).astype(jnp.float32)
    pooled_acc[...] += jnp.dot(oh, h, preferred_element_type=jnp.float32)
    cnt_acc[...] += jnp.broadcast_to(
        jnp.sum(oh, axis=1, keepdims=True), (G, HID)
    )

    @pl.when(r == NG - 1)
    def _fin():
        cnt = cnt_acc[...]
        sums = pooled_acc[...] + cnt * b_ref[...]
        mean = sums / jnp.maximum(cnt, 1.0)
        out_ref[...] = (
            jnp.dot(mean, wc_ref[...], preferred_element_type=jnp.float32)
            + bc_ref[...]
        )


def _final_call(acc2, deg_t, b2_2d, batch_r, Wc_p, bc_p):
    return pl.pallas_call(
        _final_body,
        grid=(NG,),
        in_specs=[
            pl.BlockSpec((NC, TR, HALF), lambda r: (0, r, 0)),
            pl.BlockSpec((TR, NC), lambda r: (r, 0)),
            pl.BlockSpec((1, HID), lambda r: (0, 0)),
            pl.BlockSpec((1, 1, TR), lambda r: (r, 0, 0)),
            pl.BlockSpec((HID, 128), lambda r: (0, 0)),
            pl.BlockSpec((1, 128), lambda r: (0, 0)),
        ],
        out_specs=pl.BlockSpec((G, 128), lambda r: (0, 0)),
        out_shape=jax.ShapeDtypeStruct((G, 128), jnp.float32),
        scratch_shapes=[
            pltpu.VMEM((G, HID), jnp.float32),
            pltpu.VMEM((G, HID), jnp.float32),
        ],
    )(acc2, deg_t, b2_2d, batch_r, Wc_p, bc_p)


def kernel(x, edge_index, batch, W1, b1, W2, b2, Wc, bc):
    x_p = jnp.pad(x, ((0, NPAD - N), (0, 0)))
    src = jnp.pad(edge_index[0], (0, EP - E))
    dst = jnp.pad(edge_index[1], (0, EP - E), constant_values=N)
    src_r = src.reshape(NS, NCH, CH)
    dst_r = dst.reshape(NS, NCH, CH)
    dst_d = dst.reshape(NC, NS, ED)

    deg_p = _deg_call(dst_d)
    deg_t = deg_p.T

    u1 = _mm1_call(x_p, W1, deg_t)
    acc1 = _agg_call(src_r, dst_r, u1)
    u2 = _mm2_call(acc1, W2, jnp.reshape(b1, (1, HID)), deg_t)
    acc2 = _agg_call(src_r, dst_r, u2)

    batch_r = jnp.pad(batch, (0, NPAD - N), constant_values=G).reshape(NG, 1, TR)
    Wc_p = jnp.pad(Wc, ((0, 0), (0, 128 - C)))
    bc_p = jnp.reshape(jnp.pad(bc, (0, 128 - C)), (1, 128))

    out_p = _final_call(acc2, deg_t, jnp.reshape(b2, (1, HID)), batch_r, Wc_p, bc_p)
    return out_p[:, :C]
